# baseline (speedup 1.0000x reference)
import functools

import jax
import jax.numpy as jnp
from jax import lax
from jax.experimental import pallas as pl
from jax.experimental.pallas import tpu as pltpu
from jax.experimental.pallas import tpu_sc as plsc

N = 10000
E = 320000
D = 128
G = 64

NC = 2
NS = 16
NW = NC * NS
CHUNK = 128
EPW = E // NW
FULL = EPW // CHUNK
TAIL = EPW - FULL * CHUNK
CP_A = 632
CP_B = 576
CP_SPLIT = 14 * CP_A
ZROWS = 40
DEG_PAD = 10240


def _sc_degree(dst):
    mesh = plsc.VectorSubcoreMesh(
        core_axis_name="c", subcore_axis_name="s", num_cores=NC, num_subcores=NS
    )

    @functools.partial(
        pl.kernel,
        out_type=jax.ShapeDtypeStruct((NC, DEG_PAD), jnp.float32),
        mesh=mesh,
        scratch_types=[
            pltpu.VMEM((CHUNK,), jnp.int32),
            pltpu.VMEM((CHUNK,), jnp.int32),
            pltpu.VMEM((CHUNK,), jnp.int32),
            pltpu.VMEM((TAIL,), jnp.int32),
            pltpu.VMEM((CHUNK,), jnp.float32),
            pltpu.VMEM((640,), jnp.float32),
            pltpu.VMEM_SHARED((DEG_PAD,), jnp.float32),
            pltpu.SemaphoreType.DMA,
            pltpu.SemaphoreType.DMA,
            pltpu.SemaphoreType.DMA,
            pltpu.SemaphoreType.DMA,
            pltpu.SemaphoreType.DMA,
            pltpu.SemaphoreType.DMA,
        ],
    )
    def deg_kernel(dst_hbm, out_hbm, didx0, didx1, didx2, didxt, ones, zbuf,
                   acc, si0, si1, si2, ss0, ss1, ss2):
        cid = lax.axis_index("c")
        sid = lax.axis_index("s")
        wid = sid * NC + cid
        base = wid * EPW
        dbuf = (didx0, didx1, didx2)
        isems = (si0, si1, si2)
        ssems = (ss0, ss1, ss2)

        def load_idx(i, p):
            pltpu.async_copy(
                dst_hbm.at[pl.ds(base + i * CHUNK, CHUNK)], dbuf[p], isems[p]
            )

        def wait_idx(p):
            pltpu.make_async_copy(
                dst_hbm.at[pl.ds(base, CHUNK)], dbuf[p], isems[p]
            ).wait()

        def wait_scatter(p):
            pltpu.make_async_copy(ones, acc.at[dbuf[p]], ssems[p]).wait()

        load_idx(0, 0)
        load_idx(1, 1)

        def fill_z(i, _):
            zbuf[pl.ds(i * 16, 16)] = jnp.zeros((16,), jnp.float32)
            return ()

        lax.fori_loop(0, 640 // 16, fill_z, ())
        for j in range(CHUNK // 16):
            ones[pl.ds(j * 16, 16)] = jnp.full((16,), 1.0, jnp.float32)
        pltpu.sync_copy(zbuf, acc.at[pl.ds(sid * 640, 640)])
        plsc.subcore_barrier()

        def step(i, p, wait_prev, prefetch):
            r = (p + 2) % 3
            if wait_prev:
                wait_scatter(r)
            if prefetch:
                load_idx(i + 2, r)
            wait_idx(p)
            pltpu.async_copy(ones, acc.at[dbuf[p]], ssems[p], add=True)

        def body(i3, _):
            step(3 * i3, 0, True, True)
            step(3 * i3 + 1, 1, True, True)
            step(3 * i3 + 2, 2, True, True)
            return ()

        step(0, 0, False, True)
        step(1, 1, True, True)
        step(2, 2, True, True)
        lax.fori_loop(1, FULL // 3 - 1, body, ())
        step(FULL - 3, 0, True, True)
        step(FULL - 2, 1, True, False)
        step(FULL - 1, 2, True, False)
        wait_scatter(2)
        pltpu.sync_copy(dst_hbm.at[pl.ds(base + FULL * CHUNK, TAIL)], didxt)
        pltpu.sync_copy(ones.at[pl.ds(0, TAIL)], acc.at[didxt], add=True)

        plsc.subcore_barrier()
        pltpu.sync_copy(
            acc.at[pl.ds(sid * 640, 640)], out_hbm.at[cid, pl.ds(sid * 640, 640)]
        )

    return deg_kernel(dst)


def _sc_spmm(h, src, dst, zrows):
    mesh = plsc.VectorSubcoreMesh(
        core_axis_name="c", subcore_axis_name="s", num_cores=NC, num_subcores=NS
    )

    @functools.partial(
        pl.kernel,
        out_type=jax.ShapeDtypeStruct((NC, N, D), jnp.float32),
        mesh=mesh,
        scratch_types=[
            pltpu.VMEM((CHUNK,), jnp.int32),
            pltpu.VMEM((CHUNK,), jnp.int32),
            pltpu.VMEM((CHUNK,), jnp.int32),
            pltpu.VMEM((CHUNK,), jnp.int32),
            pltpu.VMEM((CHUNK,), jnp.int32),
            pltpu.VMEM((CHUNK,), jnp.int32),
            pltpu.VMEM((TAIL,), jnp.int32),
            pltpu.VMEM((TAIL,), jnp.int32),
            pltpu.VMEM((CHUNK, D), jnp.float32),
            pltpu.VMEM((CHUNK, D), jnp.float32),
            pltpu.VMEM((CHUNK, D), jnp.float32),
            pltpu.VMEM_SHARED((N, D), jnp.float32),
            pltpu.SemaphoreType.DMA,
            pltpu.SemaphoreType.DMA,
            pltpu.SemaphoreType.DMA,
            pltpu.SemaphoreType.DMA,
            pltpu.SemaphoreType.DMA,
            pltpu.SemaphoreType.DMA,
            pltpu.SemaphoreType.DMA,
            pltpu.SemaphoreType.DMA,
            pltpu.SemaphoreType.DMA,
        ],
    )
    def spmm_kernel(h_hbm, src_hbm, dst_hbm, z_hbm, out_hbm, sidx0, sidx1,
                    sidx2, didx0, didx1, didx2, sidxt, didxt, rows0, rows1,
                    rows2, acc, sg0, sg1, sg2, si0, si1, si2, ss0, ss1, ss2):
        cid = lax.axis_index("c")
        sid = lax.axis_index("s")
        wid = sid * NC + cid
        base = wid * EPW
        sbuf = (sidx0, sidx1, sidx2)
        dbuf = (didx0, didx1, didx2)
        rbuf = (rows0, rows1, rows2)
        isems = (si0, si1, si2)
        gsems = (sg0, sg1, sg2)
        ssems = (ss0, ss1, ss2)

        def load_idx(i, p):
            off = base + i * CHUNK
            pltpu.async_copy(src_hbm.at[pl.ds(off, CHUNK)], sbuf[p], isems[p])
            pltpu.async_copy(dst_hbm.at[pl.ds(off, CHUNK)], dbuf[p], isems[p])

        def wait_idx(p):
            pltpu.make_async_copy(
                src_hbm.at[pl.ds(base, CHUNK)], sbuf[p], isems[p]
            ).wait()
            pltpu.make_async_copy(
                dst_hbm.at[pl.ds(base, CHUNK)], dbuf[p], isems[p]
            ).wait()

        def start_gather(p):
            pltpu.async_copy(h_hbm.at[sbuf[p]], rbuf[p], gsems[p])

        def wait_gather(p):
            pltpu.make_async_copy(
                h_hbm.at[pl.ds(0, CHUNK), :], rbuf[p], gsems[p]
            ).wait()

        def wait_scatter(p):
            pltpu.make_async_copy(rbuf[p], acc.at[dbuf[p]], ssems[p]).wait()

        load_idx(0, 0)
        load_idx(1, 1)

        @pl.when(sid < 14)
        def _zero_a():
            pltpu.sync_copy(
                z_hbm.at[pl.ds(0, CP_A), :], acc.at[pl.ds(sid * CP_A, CP_A), :]
            )

        @pl.when(sid >= 14)
        def _zero_b():
            pltpu.sync_copy(
                z_hbm.at[pl.ds(0, CP_B), :],
                acc.at[pl.ds(CP_SPLIT + (sid - 14) * CP_B, CP_B), :],
            )

        wait_idx(0)
        start_gather(0)
        plsc.subcore_barrier()

        def step(i, p, wait_prev, prefetch_idx, launch):
            q = (p + 1) % 3
            r = (p + 2) % 3
            if wait_prev:
                wait_scatter(r)
            if prefetch_idx:
                load_idx(i + 2, r)
            if launch:
                wait_idx(q)
                start_gather(q)
            wait_gather(p)
            pltpu.async_copy(rbuf[p], acc.at[dbuf[p]], ssems[p], add=True)

        def body(i3, _):
            step(3 * i3, 0, True, True, True)
            step(3 * i3 + 1, 1, True, True, True)
            step(3 * i3 + 2, 2, True, True, True)
            return ()

        step(0, 0, False, True, True)
        step(1, 1, True, True, True)
        step(2, 2, True, True, True)
        lax.fori_loop(1, FULL // 3 - 1, body, ())
        step(FULL - 3, 0, True, True, True)
        step(FULL - 2, 1, True, False, True)
        step(FULL - 1, 2, True, False, False)
        wait_scatter(2)
        pltpu.sync_copy(src_hbm.at[pl.ds(base + FULL * CHUNK, TAIL)], sidxt)
        pltpu.sync_copy(dst_hbm.at[pl.ds(base + FULL * CHUNK, TAIL)], didxt)
        pltpu.async_copy(h_hbm.at[sidxt], rows0.at[pl.ds(0, TAIL), :], sg0).wait()
        pltpu.sync_copy(rows0.at[pl.ds(0, TAIL), :], acc.at[didxt], add=True)
        plsc.subcore_barrier()

        @pl.when(sid < 14)
        def _copy_out_a():
            pltpu.sync_copy(
                acc.at[pl.ds(sid * CP_A, CP_A), :],
                out_hbm.at[cid, pl.ds(sid * CP_A, CP_A), :],
            )

        @pl.when(sid >= 14)
        def _copy_out_b():
            r0 = CP_SPLIT + (sid - 14) * CP_B
            pltpu.sync_copy(
                acc.at[pl.ds(r0, CP_B), :],
                out_hbm.at[cid, pl.ds(r0, CP_B), :],
            )

    return spmm_kernel(h, src, dst, zrows)


def _tc_pre_kernel(x_ref, w_ref, da_ref, db_ref, h_ref, dis_ref):
    dis = lax.rsqrt(da_ref[...] + db_ref[...] + 1.0)
    dis_ref[...] = dis
    h_ref[...] = (
        jnp.dot(x_ref[...], w_ref[...], preferred_element_type=jnp.float32) * dis
    )


def _tc_mid_kernel(agg_ref, hp_ref, dis_ref, b_ref, w_ref, out_ref):
    dis = dis_ref[...]
    s = jnp.maximum(dis * (agg_ref[0] + agg_ref[1] + hp_ref[...]) + b_ref[...], 0.0)
    out_ref[...] = (
        jnp.dot(s, w_ref[...], preferred_element_type=jnp.float32) * dis
    )


def _tc_fin_kernel(agg_ref, hp_ref, dis_ref, b_ref, batch_ref, wfc_ref,
                   bfc_ref, out_ref):
    dis = dis_ref[...]
    s = jnp.maximum(dis * (agg_ref[0] + agg_ref[1] + hp_ref[...]) + b_ref[...], 0.0)
    gids = lax.broadcasted_iota(jnp.int32, (G, N), 0)
    onehot = jnp.where(gids == batch_ref[...], 1.0, 0.0)
    sums = jnp.dot(onehot, s, preferred_element_type=jnp.float32)
    counts = jnp.sum(onehot, axis=1, keepdims=True)
    pooled = sums / jnp.maximum(counts, 1.0)
    logits = jnp.dot(pooled, wfc_ref[...], preferred_element_type=jnp.float32)
    out_ref[...] = jax.nn.sigmoid(logits + bfc_ref[...])


def kernel(x, edge_index, batch, W1, b1, W2, b2, Wfc, bfc):
    src = edge_index[0]
    dst = edge_index[1]

    degp = _sc_degree(dst)
    dega = degp[0, :N].reshape(N, 1)
    degb = degp[1, :N].reshape(N, 1)

    h1p, dis = pl.pallas_call(
        _tc_pre_kernel,
        out_shape=(
            jax.ShapeDtypeStruct((N, D), jnp.float32),
            jax.ShapeDtypeStruct((N, 1), jnp.float32),
        ),
    )(x, W1, dega, degb)

    zrows = jnp.zeros((CP_A, D), jnp.float32)
    agg1 = _sc_spmm(h1p, src, dst, zrows)

    h2p = pl.pallas_call(
        _tc_mid_kernel,
        out_shape=jax.ShapeDtypeStruct((N, D), jnp.float32),
    )(agg1, h1p, dis, b1.reshape(1, D), W2)

    agg2 = _sc_spmm(h2p, src, dst, zrows)

    out = pl.pallas_call(
        _tc_fin_kernel,
        out_shape=jax.ShapeDtypeStruct((G, 1), jnp.float32),
    )(agg2, h2p, dis, b2.reshape(1, D), batch.reshape(1, N),
      Wfc, bfc.reshape(1, 1))
    return out


# speedup vs baseline: 1.0172x; 1.0172x over previous
import functools

import jax
import jax.numpy as jnp
from jax import lax
from jax.experimental import pallas as pl
from jax.experimental.pallas import tpu as pltpu
from jax.experimental.pallas import tpu_sc as plsc

N = 10000
E = 320000
D = 128
G = 64

NC = 2
NS = 16
NW = NC * NS
CHUNK = 128
EPW = E // NW
FULL = EPW // CHUNK
TAIL = EPW - FULL * CHUNK
COPY_TILES = 10
ROWS_PER_TILE = N // COPY_TILES
ZROWS = 40
DEG_PAD = 10240


def _sc_degree(dst):
    mesh = plsc.VectorSubcoreMesh(
        core_axis_name="c", subcore_axis_name="s", num_cores=NC, num_subcores=NS
    )

    @functools.partial(
        pl.kernel,
        out_type=jax.ShapeDtypeStruct((NC, DEG_PAD), jnp.float32),
        mesh=mesh,
        scratch_types=[
            pltpu.VMEM((CHUNK,), jnp.int32),
            pltpu.VMEM((CHUNK,), jnp.int32),
            pltpu.VMEM((CHUNK,), jnp.int32),
            pltpu.VMEM((TAIL,), jnp.int32),
            pltpu.VMEM((CHUNK,), jnp.float32),
            pltpu.VMEM((640,), jnp.float32),
            pltpu.VMEM_SHARED((DEG_PAD,), jnp.float32),
            pltpu.SemaphoreType.DMA,
            pltpu.SemaphoreType.DMA,
            pltpu.SemaphoreType.DMA,
            pltpu.SemaphoreType.DMA,
            pltpu.SemaphoreType.DMA,
            pltpu.SemaphoreType.DMA,
        ],
    )
    def deg_kernel(dst_hbm, out_hbm, didx0, didx1, didx2, didxt, ones, zbuf,
                   acc, si0, si1, si2, ss0, ss1, ss2):
        cid = lax.axis_index("c")
        sid = lax.axis_index("s")
        wid = sid * NC + cid
        base = wid * EPW
        dbuf = (didx0, didx1, didx2)
        isems = (si0, si1, si2)
        ssems = (ss0, ss1, ss2)

        def load_idx(i, p):
            pltpu.async_copy(
                dst_hbm.at[pl.ds(base + i * CHUNK, CHUNK)], dbuf[p], isems[p]
            )

        def wait_idx(p):
            pltpu.make_async_copy(
                dst_hbm.at[pl.ds(base, CHUNK)], dbuf[p], isems[p]
            ).wait()

        def wait_scatter(p):
            pltpu.make_async_copy(ones, acc.at[dbuf[p]], ssems[p]).wait()

        load_idx(0, 0)
        load_idx(1, 1)

        def fill_z(i, _):
            zbuf[pl.ds(i * 16, 16)] = jnp.zeros((16,), jnp.float32)
            return ()

        lax.fori_loop(0, 640 // 16, fill_z, ())
        for j in range(CHUNK // 16):
            ones[pl.ds(j * 16, 16)] = jnp.full((16,), 1.0, jnp.float32)
        pltpu.sync_copy(zbuf, acc.at[pl.ds(sid * 640, 640)])
        plsc.subcore_barrier()

        def step(i, p, wait_prev, prefetch):
            r = (p + 2) % 3
            if wait_prev:
                wait_scatter(r)
            if prefetch:
                load_idx(i + 2, r)
            wait_idx(p)
            pltpu.async_copy(ones, acc.at[dbuf[p]], ssems[p], add=True)

        def body(i3, _):
            step(3 * i3, 0, True, True)
            step(3 * i3 + 1, 1, True, True)
            step(3 * i3 + 2, 2, True, True)
            return ()

        step(0, 0, False, True)
        step(1, 1, True, True)
        step(2, 2, True, True)
        lax.fori_loop(1, FULL // 3 - 1, body, ())
        step(FULL - 3, 0, True, True)
        step(FULL - 2, 1, True, False)
        step(FULL - 1, 2, True, False)
        wait_scatter(2)
        pltpu.sync_copy(dst_hbm.at[pl.ds(base + FULL * CHUNK, TAIL)], didxt)
        pltpu.sync_copy(ones.at[pl.ds(0, TAIL)], acc.at[didxt], add=True)

        plsc.subcore_barrier()
        pltpu.sync_copy(
            acc.at[pl.ds(sid * 640, 640)], out_hbm.at[cid, pl.ds(sid * 640, 640)]
        )

    return deg_kernel(dst)


def _sc_spmm(h, src, dst, zrows):
    mesh = plsc.VectorSubcoreMesh(
        core_axis_name="c", subcore_axis_name="s", num_cores=NC, num_subcores=NS
    )

    @functools.partial(
        pl.kernel,
        out_type=jax.ShapeDtypeStruct((NC, N, D), jnp.float32),
        mesh=mesh,
        scratch_types=[
            pltpu.VMEM((CHUNK,), jnp.int32),
            pltpu.VMEM((CHUNK,), jnp.int32),
            pltpu.VMEM((CHUNK,), jnp.int32),
            pltpu.VMEM((CHUNK,), jnp.int32),
            pltpu.VMEM((CHUNK,), jnp.int32),
            pltpu.VMEM((CHUNK,), jnp.int32),
            pltpu.VMEM((TAIL,), jnp.int32),
            pltpu.VMEM((TAIL,), jnp.int32),
            pltpu.VMEM((CHUNK, D), jnp.float32),
            pltpu.VMEM((CHUNK, D), jnp.float32),
            pltpu.VMEM((CHUNK, D), jnp.float32),
            pltpu.VMEM_SHARED((N, D), jnp.float32),
            pltpu.SemaphoreType.DMA,
            pltpu.SemaphoreType.DMA,
            pltpu.SemaphoreType.DMA,
            pltpu.SemaphoreType.DMA,
            pltpu.SemaphoreType.DMA,
            pltpu.SemaphoreType.DMA,
            pltpu.SemaphoreType.DMA,
            pltpu.SemaphoreType.DMA,
            pltpu.SemaphoreType.DMA,
        ],
    )
    def spmm_kernel(h_hbm, src_hbm, dst_hbm, z_hbm, out_hbm, sidx0, sidx1,
                    sidx2, didx0, didx1, didx2, sidxt, didxt, rows0, rows1,
                    rows2, acc, sg0, sg1, sg2, si0, si1, si2, ss0, ss1, ss2):
        cid = lax.axis_index("c")
        sid = lax.axis_index("s")
        wid = sid * NC + cid
        base = wid * EPW
        sbuf = (sidx0, sidx1, sidx2)
        dbuf = (didx0, didx1, didx2)
        rbuf = (rows0, rows1, rows2)
        isems = (si0, si1, si2)
        gsems = (sg0, sg1, sg2)
        ssems = (ss0, ss1, ss2)

        def load_idx(i, p):
            off = base + i * CHUNK
            pltpu.async_copy(src_hbm.at[pl.ds(off, CHUNK)], sbuf[p], isems[p])
            pltpu.async_copy(dst_hbm.at[pl.ds(off, CHUNK)], dbuf[p], isems[p])

        def wait_idx(p):
            pltpu.make_async_copy(
                src_hbm.at[pl.ds(base, CHUNK)], sbuf[p], isems[p]
            ).wait()
            pltpu.make_async_copy(
                dst_hbm.at[pl.ds(base, CHUNK)], dbuf[p], isems[p]
            ).wait()

        def start_gather(p):
            pltpu.async_copy(h_hbm.at[sbuf[p]], rbuf[p], gsems[p])

        def wait_gather(p):
            pltpu.make_async_copy(
                h_hbm.at[pl.ds(0, CHUNK), :], rbuf[p], gsems[p]
            ).wait()

        def wait_scatter(p):
            pltpu.make_async_copy(rbuf[p], acc.at[dbuf[p]], ssems[p]).wait()

        load_idx(0, 0)
        load_idx(1, 1)

        @pl.when(sid < COPY_TILES)
        def _zero():
            pltpu.sync_copy(z_hbm, acc.at[pl.ds(sid * ROWS_PER_TILE, ROWS_PER_TILE), :])

        wait_idx(0)
        start_gather(0)
        plsc.subcore_barrier()

        def step(i, p, wait_prev, prefetch_idx, launch):
            q = (p + 1) % 3
            r = (p + 2) % 3
            if wait_prev:
                wait_scatter(r)
            if prefetch_idx:
                load_idx(i + 2, r)
            if launch:
                wait_idx(q)
                start_gather(q)
            wait_gather(p)
            pltpu.async_copy(rbuf[p], acc.at[dbuf[p]], ssems[p], add=True)

        def body(i3, _):
            step(3 * i3, 0, True, True, True)
            step(3 * i3 + 1, 1, True, True, True)
            step(3 * i3 + 2, 2, True, True, True)
            return ()

        step(0, 0, False, True, True)
        step(1, 1, True, True, True)
        step(2, 2, True, True, True)
        lax.fori_loop(1, FULL // 3 - 1, body, ())
        step(FULL - 3, 0, True, True, True)
        step(FULL - 2, 1, True, False, True)
        step(FULL - 1, 2, True, False, False)
        wait_scatter(2)
        pltpu.sync_copy(src_hbm.at[pl.ds(base + FULL * CHUNK, TAIL)], sidxt)
        pltpu.sync_copy(dst_hbm.at[pl.ds(base + FULL * CHUNK, TAIL)], didxt)
        pltpu.async_copy(h_hbm.at[sidxt], rows0.at[pl.ds(0, TAIL), :], sg0).wait()
        pltpu.sync_copy(rows0.at[pl.ds(0, TAIL), :], acc.at[didxt], add=True)
        plsc.subcore_barrier()

        @pl.when(sid < COPY_TILES)
        def _copy_out():
            pltpu.sync_copy(
                acc.at[pl.ds(sid * ROWS_PER_TILE, ROWS_PER_TILE), :],
                out_hbm.at[cid, pl.ds(sid * ROWS_PER_TILE, ROWS_PER_TILE), :],
            )

    return spmm_kernel(h, src, dst, zrows)


def _tc_pre_kernel(x_ref, w_ref, da_ref, db_ref, h_ref, dis_ref):
    dis = lax.rsqrt(da_ref[...] + db_ref[...] + 1.0)
    dis_ref[...] = dis
    h_ref[...] = (
        jnp.dot(x_ref[...], w_ref[...], preferred_element_type=jnp.float32) * dis
    )


def _tc_mid_kernel(agg_ref, hp_ref, dis_ref, b_ref, w_ref, out_ref):
    dis = dis_ref[...]
    s = jnp.maximum(dis * (agg_ref[0] + agg_ref[1] + hp_ref[...]) + b_ref[...], 0.0)
    out_ref[...] = (
        jnp.dot(s, w_ref[...], preferred_element_type=jnp.float32) * dis
    )


def _tc_fin_kernel(agg_ref, hp_ref, dis_ref, b_ref, batch_ref, wfc_ref,
                   bfc_ref, out_ref):
    dis = dis_ref[...]
    s = jnp.maximum(dis * (agg_ref[0] + agg_ref[1] + hp_ref[...]) + b_ref[...], 0.0)
    gids = lax.broadcasted_iota(jnp.int32, (G, N), 0)
    onehot = jnp.where(gids == batch_ref[...], 1.0, 0.0)
    sums = jnp.dot(onehot, s, preferred_element_type=jnp.float32)
    counts = jnp.sum(onehot, axis=1, keepdims=True)
    pooled = sums / jnp.maximum(counts, 1.0)
    logits = jnp.dot(pooled, wfc_ref[...], preferred_element_type=jnp.float32)
    out_ref[...] = jax.nn.sigmoid(logits + bfc_ref[...])


def kernel(x, edge_index, batch, W1, b1, W2, b2, Wfc, bfc):
    src = edge_index[0]
    dst = edge_index[1]

    degp = _sc_degree(dst)
    dega = degp[0, :N].reshape(N, 1)
    degb = degp[1, :N].reshape(N, 1)

    h1p, dis = pl.pallas_call(
        _tc_pre_kernel,
        out_shape=(
            jax.ShapeDtypeStruct((N, D), jnp.float32),
            jax.ShapeDtypeStruct((N, 1), jnp.float32),
        ),
    )(x, W1, dega, degb)

    zrows = jnp.zeros((ROWS_PER_TILE, D), jnp.float32)
    agg1 = _sc_spmm(h1p, src, dst, zrows)

    h2p = pl.pallas_call(
        _tc_mid_kernel,
        out_shape=jax.ShapeDtypeStruct((N, D), jnp.float32),
    )(agg1, h1p, dis, b1.reshape(1, D), W2)

    agg2 = _sc_spmm(h2p, src, dst, zrows)

    out = pl.pallas_call(
        _tc_fin_kernel,
        out_shape=jax.ShapeDtypeStruct((G, 1), jnp.float32),
    )(agg2, h2p, dis, b2.reshape(1, D), batch.reshape(1, N),
      Wfc, bfc.reshape(1, 1))
    return out


# speedup vs baseline: 1.0334x; 1.0159x over previous
import functools

import jax
import jax.numpy as jnp
from jax import lax
from jax.experimental import pallas as pl
from jax.experimental.pallas import tpu as pltpu
from jax.experimental.pallas import tpu_sc as plsc

N = 10000
E = 320000
D = 128
G = 64

NC = 2
NS = 16
NW = NC * NS
CHUNK = 128
EPW = E // NW
FULL = EPW // CHUNK
TAIL = EPW - FULL * CHUNK
COPY_TILES = 10
ROWS_PER_TILE = N // COPY_TILES
ZROWS = 40
DEG_PAD = 10240


def _sc_degree(dst):
    mesh = plsc.VectorSubcoreMesh(
        core_axis_name="c", subcore_axis_name="s", num_cores=NC, num_subcores=NS
    )

    @functools.partial(
        pl.kernel,
        out_type=jax.ShapeDtypeStruct((NC, DEG_PAD), jnp.float32),
        mesh=mesh,
        scratch_types=[
            pltpu.VMEM((CHUNK,), jnp.int32),
            pltpu.VMEM((CHUNK,), jnp.int32),
            pltpu.VMEM((CHUNK,), jnp.int32),
            pltpu.VMEM((TAIL,), jnp.int32),
            pltpu.VMEM((CHUNK,), jnp.float32),
            pltpu.VMEM((640,), jnp.float32),
            pltpu.VMEM_SHARED((DEG_PAD,), jnp.float32),
            pltpu.SemaphoreType.DMA,
            pltpu.SemaphoreType.DMA,
            pltpu.SemaphoreType.DMA,
            pltpu.SemaphoreType.DMA,
            pltpu.SemaphoreType.DMA,
            pltpu.SemaphoreType.DMA,
        ],
    )
    def deg_kernel(dst_hbm, out_hbm, didx0, didx1, didx2, didxt, ones, zbuf,
                   acc, si0, si1, si2, ss0, ss1, ss2):
        cid = lax.axis_index("c")
        sid = lax.axis_index("s")
        wid = sid * NC + cid
        base = wid * EPW
        dbuf = (didx0, didx1, didx2)
        isems = (si0, si1, si2)
        ssems = (ss0, ss1, ss2)

        def load_idx(i, p):
            pltpu.async_copy(
                dst_hbm.at[pl.ds(base + i * CHUNK, CHUNK)], dbuf[p], isems[p]
            )

        def wait_idx(p):
            pltpu.make_async_copy(
                dst_hbm.at[pl.ds(base, CHUNK)], dbuf[p], isems[p]
            ).wait()

        def wait_scatter(p):
            pltpu.make_async_copy(ones, acc.at[dbuf[p]], ssems[p]).wait()

        load_idx(0, 0)
        load_idx(1, 1)

        def fill_z(i, _):
            zbuf[pl.ds(i * 16, 16)] = jnp.zeros((16,), jnp.float32)
            return ()

        lax.fori_loop(0, 640 // 16, fill_z, ())
        for j in range(CHUNK // 16):
            ones[pl.ds(j * 16, 16)] = jnp.full((16,), 1.0, jnp.float32)
        pltpu.sync_copy(zbuf, acc.at[pl.ds(sid * 640, 640)])
        plsc.subcore_barrier()

        def step(i, p, wait_prev, prefetch):
            r = (p + 2) % 3
            if wait_prev:
                wait_scatter(r)
            if prefetch:
                load_idx(i + 2, r)
            wait_idx(p)
            pltpu.async_copy(ones, acc.at[dbuf[p]], ssems[p], add=True)

        def body(i3, _):
            step(3 * i3, 0, True, True)
            step(3 * i3 + 1, 1, True, True)
            step(3 * i3 + 2, 2, True, True)
            return ()

        step(0, 0, False, True)
        step(1, 1, True, True)
        step(2, 2, True, True)
        lax.fori_loop(1, FULL // 3 - 1, body, ())
        step(FULL - 3, 0, True, True)
        step(FULL - 2, 1, True, False)
        step(FULL - 1, 2, True, False)
        wait_scatter(2)
        pltpu.sync_copy(dst_hbm.at[pl.ds(base + FULL * CHUNK, TAIL)], didxt)
        pltpu.sync_copy(ones.at[pl.ds(0, TAIL)], acc.at[didxt], add=True)

        plsc.subcore_barrier()
        pltpu.sync_copy(
            acc.at[pl.ds(sid * 640, 640)], out_hbm.at[cid, pl.ds(sid * 640, 640)]
        )

    return deg_kernel(dst)


def _sc_spmm(h, src, dst, zrows):
    mesh = plsc.VectorSubcoreMesh(
        core_axis_name="c", subcore_axis_name="s", num_cores=NC, num_subcores=NS
    )

    @functools.partial(
        pl.kernel,
        out_type=jax.ShapeDtypeStruct((NC, N, D), jnp.float32),
        mesh=mesh,
        scratch_types=[
            pltpu.VMEM((CHUNK,), jnp.int32),
            pltpu.VMEM((CHUNK,), jnp.int32),
            pltpu.VMEM((CHUNK,), jnp.int32),
            pltpu.VMEM((CHUNK,), jnp.int32),
            pltpu.VMEM((CHUNK,), jnp.int32),
            pltpu.VMEM((CHUNK,), jnp.int32),
            pltpu.VMEM((TAIL,), jnp.int32),
            pltpu.VMEM((TAIL,), jnp.int32),
            pltpu.VMEM((CHUNK, D), jnp.float32),
            pltpu.VMEM((CHUNK, D), jnp.float32),
            pltpu.VMEM((CHUNK, D), jnp.float32),
            pltpu.VMEM_SHARED((N, D), jnp.float32),
            pltpu.SemaphoreType.DMA,
            pltpu.SemaphoreType.DMA,
            pltpu.SemaphoreType.DMA,
            pltpu.SemaphoreType.DMA,
            pltpu.SemaphoreType.DMA,
            pltpu.SemaphoreType.DMA,
            pltpu.SemaphoreType.DMA,
            pltpu.SemaphoreType.DMA,
            pltpu.SemaphoreType.DMA,
        ],
    )
    def spmm_kernel(h_hbm, src_hbm, dst_hbm, z_hbm, out_hbm, sidx0, sidx1,
                    sidx2, didx0, didx1, didx2, sidxt, didxt, rows0, rows1,
                    rows2, acc, sg0, sg1, sg2, si0, si1, si2, ss0, ss1, ss2):
        cid = lax.axis_index("c")
        sid = lax.axis_index("s")
        wid = sid * NC + cid
        base = wid * EPW
        sbuf = (sidx0, sidx1, sidx2)
        dbuf = (didx0, didx1, didx2)
        rbuf = (rows0, rows1, rows2)
        isems = (si0, si1, si2)
        gsems = (sg0, sg1, sg2)
        ssems = (ss0, ss1, ss2)

        def load_idx(i, p):
            off = base + i * CHUNK
            pltpu.async_copy(src_hbm.at[pl.ds(off, CHUNK)], sbuf[p], isems[p])
            pltpu.async_copy(dst_hbm.at[pl.ds(off, CHUNK)], dbuf[p], isems[p])

        def wait_idx(p):
            pltpu.make_async_copy(
                src_hbm.at[pl.ds(base, CHUNK)], sbuf[p], isems[p]
            ).wait()
            pltpu.make_async_copy(
                dst_hbm.at[pl.ds(base, CHUNK)], dbuf[p], isems[p]
            ).wait()

        def start_gather(p):
            pltpu.async_copy(h_hbm.at[sbuf[p]], rbuf[p], gsems[p])

        def wait_gather(p):
            pltpu.make_async_copy(
                h_hbm.at[pl.ds(0, CHUNK), :], rbuf[p], gsems[p]
            ).wait()

        def wait_scatter(p):
            pltpu.make_async_copy(rbuf[p], acc.at[dbuf[p]], ssems[p]).wait()

        load_idx(0, 0)
        load_idx(1, 1)

        @pl.when(sid < COPY_TILES)
        def _zero():
            pltpu.sync_copy(z_hbm, acc.at[pl.ds(sid * ROWS_PER_TILE, ROWS_PER_TILE), :])

        wait_idx(0)
        start_gather(0)
        plsc.subcore_barrier()

        def step(i, p, wait_prev, prefetch_idx, launch):
            q = (p + 1) % 3
            r = (p + 2) % 3
            if launch:
                wait_idx(q)
                start_gather(q)
            if wait_prev:
                wait_scatter(r)
            if prefetch_idx:
                load_idx(i + 2, r)
            wait_gather(p)
            pltpu.async_copy(rbuf[p], acc.at[dbuf[p]], ssems[p], add=True)

        def body(i3, _):
            step(3 * i3, 0, True, True, True)
            step(3 * i3 + 1, 1, True, True, True)
            step(3 * i3 + 2, 2, True, True, True)
            return ()

        step(0, 0, False, True, True)
        step(1, 1, True, True, True)
        step(2, 2, True, True, True)
        lax.fori_loop(1, FULL // 3 - 1, body, ())
        step(FULL - 3, 0, True, True, True)
        step(FULL - 2, 1, True, False, True)
        step(FULL - 1, 2, True, False, False)
        wait_scatter(2)
        pltpu.sync_copy(src_hbm.at[pl.ds(base + FULL * CHUNK, TAIL)], sidxt)
        pltpu.sync_copy(dst_hbm.at[pl.ds(base + FULL * CHUNK, TAIL)], didxt)
        pltpu.async_copy(h_hbm.at[sidxt], rows0.at[pl.ds(0, TAIL), :], sg0).wait()
        pltpu.sync_copy(rows0.at[pl.ds(0, TAIL), :], acc.at[didxt], add=True)
        plsc.subcore_barrier()

        @pl.when(sid < COPY_TILES)
        def _copy_out():
            pltpu.sync_copy(
                acc.at[pl.ds(sid * ROWS_PER_TILE, ROWS_PER_TILE), :],
                out_hbm.at[cid, pl.ds(sid * ROWS_PER_TILE, ROWS_PER_TILE), :],
            )

    return spmm_kernel(h, src, dst, zrows)


def _tc_pre_kernel(x_ref, w_ref, da_ref, db_ref, h_ref, dis_ref):
    dis = lax.rsqrt(da_ref[...] + db_ref[...] + 1.0)
    dis_ref[...] = dis
    h_ref[...] = (
        jnp.dot(x_ref[...], w_ref[...], preferred_element_type=jnp.float32) * dis
    )


def _tc_mid_kernel(agg_ref, hp_ref, dis_ref, b_ref, w_ref, out_ref):
    dis = dis_ref[...]
    s = jnp.maximum(dis * (agg_ref[0] + agg_ref[1] + hp_ref[...]) + b_ref[...], 0.0)
    out_ref[...] = (
        jnp.dot(s, w_ref[...], preferred_element_type=jnp.float32) * dis
    )


def _tc_fin_kernel(agg_ref, hp_ref, dis_ref, b_ref, batch_ref, wfc_ref,
                   bfc_ref, out_ref):
    dis = dis_ref[...]
    s = jnp.maximum(dis * (agg_ref[0] + agg_ref[1] + hp_ref[...]) + b_ref[...], 0.0)
    gids = lax.broadcasted_iota(jnp.int32, (G, N), 0)
    onehot = jnp.where(gids == batch_ref[...], 1.0, 0.0)
    sums = jnp.dot(onehot, s, preferred_element_type=jnp.float32)
    counts = jnp.sum(onehot, axis=1, keepdims=True)
    pooled = sums / jnp.maximum(counts, 1.0)
    logits = jnp.dot(pooled, wfc_ref[...], preferred_element_type=jnp.float32)
    out_ref[...] = jax.nn.sigmoid(logits + bfc_ref[...])


def kernel(x, edge_index, batch, W1, b1, W2, b2, Wfc, bfc):
    src = edge_index[0]
    dst = edge_index[1]

    degp = _sc_degree(dst)
    dega = degp[0, :N].reshape(N, 1)
    degb = degp[1, :N].reshape(N, 1)

    h1p, dis = pl.pallas_call(
        _tc_pre_kernel,
        out_shape=(
            jax.ShapeDtypeStruct((N, D), jnp.float32),
            jax.ShapeDtypeStruct((N, 1), jnp.float32),
        ),
    )(x, W1, dega, degb)

    zrows = jnp.zeros((ROWS_PER_TILE, D), jnp.float32)
    agg1 = _sc_spmm(h1p, src, dst, zrows)

    h2p = pl.pallas_call(
        _tc_mid_kernel,
        out_shape=jax.ShapeDtypeStruct((N, D), jnp.float32),
    )(agg1, h1p, dis, b1.reshape(1, D), W2)

    agg2 = _sc_spmm(h2p, src, dst, zrows)

    out = pl.pallas_call(
        _tc_fin_kernel,
        out_shape=jax.ShapeDtypeStruct((G, 1), jnp.float32),
    )(agg2, h2p, dis, b2.reshape(1, D), batch.reshape(1, N),
      Wfc, bfc.reshape(1, 1))
    return out
